# speedup vs baseline: 1.2317x; 1.2317x over previous
import jax
import jax.numpy as jnp
from jax.experimental import pallas as pl
from jax.experimental.pallas import tpu as pltpu


def _fused_kernel(x_ref, w0_ref, wr_ref, b_ref, wd_ref, bd_ref, out_ref,
                  g0_scr, h_scr, c_scr, acc_scr):
    t = pl.program_id(0)
    num_layers = b_ref.shape[0]
    hidden = b_ref.shape[2] // 4
    in_size = x_ref.shape[2]

    @pl.when(t == 0)
    def _init():
        xs = x_ref[:, 0, :]
        g0_scr[...] = (jnp.dot(xs, w0_ref[0:in_size, :],
                               preferred_element_type=jnp.float32)
                       + b_ref[0])
        h_scr[...] = jnp.zeros_like(h_scr)
        c_scr[...] = jnp.zeros_like(c_scr)
        acc_scr[...] = jnp.zeros_like(acc_scr)

    def step_layer(gates, l):
        i_g = jax.nn.sigmoid(gates[:, 0 * hidden:1 * hidden])
        f_g = jax.nn.sigmoid(gates[:, 1 * hidden:2 * hidden])
        g_g = jnp.tanh(gates[:, 2 * hidden:3 * hidden])
        o_g = jax.nn.sigmoid(gates[:, 3 * hidden:4 * hidden])
        c_new = f_g * c_scr[l] + i_g * g_g
        h_new = o_g * jnp.tanh(c_new)
        c_scr[l] = c_new
        h_scr[l] = h_new
        return h_new

    gates = (g0_scr[pl.ds(t, 1), :]
             + jnp.dot(h_scr[0], w0_ref[in_size:in_size + hidden, :],
                       preferred_element_type=jnp.float32))
    layer_in = step_layer(gates, 0)

    for l in range(1, num_layers):
        gates = (jnp.dot(layer_in, wr_ref[l - 1, 0:hidden, :],
                         preferred_element_type=jnp.float32)
                 + jnp.dot(h_scr[l], wr_ref[l - 1, hidden:2 * hidden, :],
                           preferred_element_type=jnp.float32)
                 + b_ref[l])
        layer_in = step_layer(gates, l)

    acc_scr[...] += jnp.dot(layer_in, wd_ref[0],
                            preferred_element_type=jnp.float32)

    @pl.when(t == pl.num_programs(0) - 1)
    def _finalize():
        out_ref[...] = jax.nn.sigmoid(acc_scr[...] + bd_ref[...]).astype(out_ref.dtype)


@jax.jit
def kernel(x, w0, wr, b_all, wd, bd):
    seq_len, _, in_size = x.shape
    num_layers = b_all.shape[0]
    hidden = b_all.shape[2] // 4
    out_size = wd.shape[2]
    lr = wr.shape[0]

    return pl.pallas_call(
        _fused_kernel,
        out_shape=jax.ShapeDtypeStruct((1, out_size), jnp.float32),
        grid_spec=pltpu.PrefetchScalarGridSpec(
            num_scalar_prefetch=0,
            grid=(seq_len,),
            in_specs=[
                pl.BlockSpec((seq_len, 1, in_size), lambda t: (0, 0, 0)),
                pl.BlockSpec((in_size + hidden, 4 * hidden), lambda t: (0, 0)),
                pl.BlockSpec((lr, 2 * hidden, 4 * hidden), lambda t: (0, 0, 0)),
                pl.BlockSpec((num_layers, 1, 4 * hidden), lambda t: (0, 0, 0)),
                pl.BlockSpec((1, hidden, out_size), lambda t: (t, 0, 0)),
                pl.BlockSpec((1, out_size), lambda t: (0, 0)),
            ],
            out_specs=pl.BlockSpec((1, out_size), lambda t: (0, 0)),
            scratch_shapes=[
                pltpu.VMEM((seq_len, 4 * hidden), jnp.float32),
                pltpu.VMEM((num_layers, 1, hidden), jnp.float32),
                pltpu.VMEM((num_layers, 1, hidden), jnp.float32),
                pltpu.VMEM((1, out_size), jnp.float32),
            ],
        ),
        compiler_params=pltpu.CompilerParams(
            dimension_semantics=("arbitrary",)),
    )(x, w0, wr, b_all, wd, bd)


# speedup vs baseline: 1.6416x; 1.3328x over previous
import jax
import jax.numpy as jnp
from jax.experimental import pallas as pl
from jax.experimental.pallas import tpu as pltpu

_CHUNK = 8


def _fused_kernel(x_ref, w0_ref, wr_ref, b_ref, wd_ref, bd_ref, out_ref,
                  g0_scr, hc_scr, h_scr, c_scr, acc_scr):
    c = pl.program_id(0)
    num_layers = b_ref.shape[0]
    hidden = b_ref.shape[2] // 4
    in_size = x_ref.shape[2]
    C = wd_ref.shape[0]

    @pl.when(c == 0)
    def _init():
        xs = x_ref[:, 0, :]
        g0_scr[...] = (jnp.dot(xs, w0_ref[0:in_size, :],
                               preferred_element_type=jnp.float32)
                       + b_ref[0])
        h_scr[...] = jnp.zeros_like(h_scr)
        c_scr[...] = jnp.zeros_like(c_scr)
        acc_scr[...] = jnp.zeros_like(acc_scr)

    def step_layer(gates, l):
        i_g = jax.nn.sigmoid(gates[:, 0 * hidden:1 * hidden])
        f_g = jax.nn.sigmoid(gates[:, 1 * hidden:2 * hidden])
        g_g = jnp.tanh(gates[:, 2 * hidden:3 * hidden])
        o_g = jax.nn.sigmoid(gates[:, 3 * hidden:4 * hidden])
        c_new = f_g * c_scr[l] + i_g * g_g
        h_new = o_g * jnp.tanh(c_new)
        c_scr[l] = c_new
        h_scr[l] = h_new
        return h_new

    base = c * C

    for j in range(C):
        gates = (g0_scr[pl.ds(base + j, 1), :]
                 + jnp.dot(h_scr[0], w0_ref[in_size:in_size + hidden, :],
                           preferred_element_type=jnp.float32))
        hc_scr[j:j + 1, :] = step_layer(gates, 0)

    for l in range(1, num_layers):
        g_chunk = (jnp.dot(hc_scr[...], wr_ref[l - 1, 0:hidden, :],
                           preferred_element_type=jnp.float32)
                   + b_ref[l])
        for j in range(C):
            gates = (g_chunk[j:j + 1, :]
                     + jnp.dot(h_scr[l], wr_ref[l - 1, hidden:2 * hidden, :],
                               preferred_element_type=jnp.float32))
            hc_scr[j:j + 1, :] = step_layer(gates, l)

    acc = acc_scr[...]
    for j in range(C):
        acc = acc + jnp.dot(hc_scr[j:j + 1, :], wd_ref[j],
                            preferred_element_type=jnp.float32)
    acc_scr[...] = acc

    @pl.when(c == pl.num_programs(0) - 1)
    def _finalize():
        out_ref[...] = jax.nn.sigmoid(acc_scr[...] + bd_ref[...]).astype(out_ref.dtype)


@jax.jit
def kernel(x, w0, wr, b_all, wd, bd):
    seq_len, _, in_size = x.shape
    num_layers = b_all.shape[0]
    hidden = b_all.shape[2] // 4
    out_size = wd.shape[2]
    lr = wr.shape[0]
    chunk = _CHUNK if seq_len % _CHUNK == 0 else 1

    return pl.pallas_call(
        _fused_kernel,
        out_shape=jax.ShapeDtypeStruct((1, out_size), jnp.float32),
        grid_spec=pltpu.PrefetchScalarGridSpec(
            num_scalar_prefetch=0,
            grid=(seq_len // chunk,),
            in_specs=[
                pl.BlockSpec((seq_len, 1, in_size), lambda c: (0, 0, 0)),
                pl.BlockSpec((in_size + hidden, 4 * hidden), lambda c: (0, 0)),
                pl.BlockSpec((lr, 2 * hidden, 4 * hidden), lambda c: (0, 0, 0)),
                pl.BlockSpec((num_layers, 1, 4 * hidden), lambda c: (0, 0, 0)),
                pl.BlockSpec((chunk, hidden, out_size), lambda c: (c, 0, 0)),
                pl.BlockSpec((1, out_size), lambda c: (0, 0)),
            ],
            out_specs=pl.BlockSpec((1, out_size), lambda c: (0, 0)),
            scratch_shapes=[
                pltpu.VMEM((seq_len, 4 * hidden), jnp.float32),
                pltpu.VMEM((chunk, hidden), jnp.float32),
                pltpu.VMEM((num_layers, 1, hidden), jnp.float32),
                pltpu.VMEM((num_layers, 1, hidden), jnp.float32),
                pltpu.VMEM((1, out_size), jnp.float32),
            ],
        ),
        compiler_params=pltpu.CompilerParams(
            dimension_semantics=("arbitrary",)),
    )(x, w0, wr, b_all, wd, bd)
